# baseline (speedup 1.0000x reference)
import jax
import jax.numpy as jnp
from jax import lax
from jax.experimental import pallas as pl
from jax.experimental.pallas import tpu as pltpu
from jax.experimental.pallas import tpu_sc as plsc

BATCH = 16384
EMB = 32
N_USERS = 1000000
N_TOURS = 100000

_info = plsc.get_sparse_core_info()
_NC = _info.num_cores
_NS = _info.num_subcores
_NW = _NC * _NS
_BPW = BATCH // _NW


def _egather_body(ug_hbm, tg_hbm, ut_hbm, tt_hbm, uout_hbm, tout_hbm,
                  ug, tg, uval, tval, sem_u, sem_t):
    wid = lax.axis_index("s") * _NC + lax.axis_index("c")
    base = wid * _BPW
    pltpu.sync_copy(ug_hbm.at[:, pl.ds(base, _BPW)], ug)
    pltpu.sync_copy(tg_hbm.at[:, pl.ds(base, _BPW)], tg)
    ucopies = [pltpu.async_copy(ut_hbm.at[ug.at[d]], uval.at[d], sem_u)
               for d in range(EMB)]
    tcopies = [pltpu.async_copy(tt_hbm.at[tg.at[d]], tval.at[d], sem_t)
               for d in range(EMB)]
    for c in ucopies:
        c.wait()
    for c in tcopies:
        c.wait()
    pltpu.sync_copy(uval, uout_hbm.at[:, pl.ds(base, _BPW)])
    pltpu.sync_copy(tval, tout_hbm.at[:, pl.ds(base, _BPW)])


def _reduce_body(u_ref, t_ref, wu_ref, wt_ref, b_ref, out_ref):
    out_ref[...] = (jnp.sum(u_ref[...] * wu_ref[...], axis=0)
                    + jnp.sum(t_ref[...] * wt_ref[...], axis=0) + b_ref[0])


@jax.jit
def kernel(users, tours, user_table, tour_table, W, b):
    users = users.astype(jnp.int32)
    tours = tours.astype(jnp.int32)
    drange = lax.iota(jnp.int32, EMB)
    ugidx = users[None, :] + (drange * N_USERS)[:, None]
    tgidx = tours[None, :] + (drange * N_TOURS)[:, None]
    utflat = user_table.T.reshape(EMB * N_USERS)
    ttflat = tour_table.T.reshape(EMB * N_TOURS)

    run = pl.kernel(
        _egather_body,
        out_type=(jax.ShapeDtypeStruct((EMB, BATCH), jnp.float32),
                  jax.ShapeDtypeStruct((EMB, BATCH), jnp.float32)),
        mesh=plsc.VectorSubcoreMesh(core_axis_name="c", subcore_axis_name="s"),
        compiler_params=pltpu.CompilerParams(
            needs_layout_passes=False, use_tc_tiling_on_sc=False),
        scratch_types=[
            pltpu.VMEM((EMB, _BPW), jnp.int32),
            pltpu.VMEM((EMB, _BPW), jnp.int32),
            pltpu.VMEM((EMB, _BPW), jnp.float32),
            pltpu.VMEM((EMB, _BPW), jnp.float32),
            pltpu.SemaphoreType.DMA,
            pltpu.SemaphoreType.DMA,
        ],
    )
    uvalT, tvalT = run(ugidx, tgidx, utflat, ttflat)

    wu = W[0, :EMB].reshape(EMB, 1)
    wt = W[0, EMB:].reshape(EMB, 1)
    out = pl.pallas_call(
        _reduce_body,
        grid=(1,),
        in_specs=[
            pl.BlockSpec((EMB, BATCH), lambda i: (0, 0)),
            pl.BlockSpec((EMB, BATCH), lambda i: (0, 0)),
            pl.BlockSpec((EMB, 1), lambda i: (0, 0)),
            pl.BlockSpec((EMB, 1), lambda i: (0, 0)),
            pl.BlockSpec(memory_space=pltpu.SMEM),
        ],
        out_specs=pl.BlockSpec((BATCH,), lambda i: (0,)),
        out_shape=jax.ShapeDtypeStruct((BATCH,), jnp.float32),
    )(uvalT, tvalT, wu, wt, b)
    return out.reshape(BATCH, 1)


# speedup vs baseline: 29.2777x; 29.2777x over previous
import jax
import jax.numpy as jnp
from jax import lax
from jax.experimental import pallas as pl
from jax.experimental.pallas import tpu as pltpu
from jax.experimental.pallas import tpu_sc as plsc

BATCH = 16384
EMB = 32
N_USERS = 1000000
N_TOURS = 100000

_info = plsc.get_sparse_core_info()
_NC = _info.num_cores
_NS = _info.num_subcores
_L = _info.num_lanes
_NW = _NC * _NS
_BPW = BATCH // _NW

_UCHUNK = 131072
_TPW = 3136
_TCH = 1024
_TLAST = N_TOURS - _TCH


def _score_body(tT_ref, w_ref, b_ref, out_ref):
    out_ref[...] = jnp.sum(tT_ref[...] * w_ref[...], axis=0) + b_ref[0]


def _scores(tT, wcol, bias, n, chunk):
    grid = (n + chunk - 1) // chunk
    return pl.pallas_call(
        _score_body,
        grid=(grid,),
        in_specs=[
            pl.BlockSpec((EMB, chunk), lambda i: (0, i)),
            pl.BlockSpec((EMB, 1), lambda i: (0, 0)),
            pl.BlockSpec(memory_space=pltpu.SMEM),
        ],
        out_specs=pl.BlockSpec((chunk,), lambda i: (i,)),
        out_shape=jax.ShapeDtypeStruct((n,), jnp.float32),
    )(tT, wcol, bias)


def _tscore_body(tt_hbm, wrep_hbm, brep_hbm, out_hbm,
                 tbuf, wrep, brep, score, sem):
    wid = lax.axis_index("s") * _NC + lax.axis_index("c")
    lo = wid * _TPW
    pltpu.sync_copy(wrep_hbm, wrep)
    pltpu.sync_copy(brep_hbm, brep)
    nch = _TPW // _TCH + 1

    def off(c):
        return jnp.minimum(lo + c * _TCH, _TLAST)

    copies = [None] * nch
    copies[0] = pltpu.async_copy(
        tt_hbm.at[:, pl.ds(off(0), _TCH)], tbuf.at[0], sem)
    for c in range(nch):
        if c + 1 < nch:
            copies[c + 1] = pltpu.async_copy(
                tt_hbm.at[:, pl.ds(off(c + 1), _TCH)],
                tbuf.at[(c + 1) % 2], sem)
        copies[c].wait()
        buf = c % 2

        def group(g, carry):
            sl = pl.ds(g * _L, _L)
            acc = brep[...]
            for d in range(EMB):
                acc = acc + tbuf[buf, d, sl] * wrep[d]
            score[sl] = acc
            return carry

        lax.fori_loop(0, _TCH // _L, group, 0)
        pltpu.sync_copy(score, out_hbm.at[pl.ds(off(c), _TCH)])


def _tscores_sc(tt2, wrep, brep):
    run = pl.kernel(
        _tscore_body,
        out_type=jax.ShapeDtypeStruct((N_TOURS,), jnp.float32),
        mesh=plsc.VectorSubcoreMesh(core_axis_name="c", subcore_axis_name="s"),
        compiler_params=pltpu.CompilerParams(
            needs_layout_passes=False, use_tc_tiling_on_sc=False),
        scratch_types=[
            pltpu.VMEM((2, EMB, _TCH), jnp.float32),
            pltpu.VMEM((EMB, _L), jnp.float32),
            pltpu.VMEM((_L,), jnp.float32),
            pltpu.VMEM((_TCH,), jnp.float32),
            pltpu.SemaphoreType.DMA,
        ],
    )
    return run(tt2, wrep, brep)


def _gather_body(users_hbm, tours_hbm, us_hbm, ts_hbm, out_hbm,
                 uidx, tidx, uval, tval, outv, sem_u, sem_t):
    wid = lax.axis_index("s") * _NC + lax.axis_index("c")
    base = wid * _BPW
    pltpu.sync_copy(users_hbm.at[pl.ds(base, _BPW)], uidx)
    pltpu.sync_copy(tours_hbm.at[pl.ds(base, _BPW)], tidx)
    cu = pltpu.async_copy(us_hbm.at[uidx], uval, sem_u)
    ct = pltpu.async_copy(ts_hbm.at[tidx], tval, sem_t)
    cu.wait()
    ct.wait()

    def group(g, carry):
        sl = pl.ds(g * _L, _L)
        outv[sl] = uval[sl] + tval[sl]
        return carry

    lax.fori_loop(0, _BPW // _L, group, 0)
    pltpu.sync_copy(outv, out_hbm.at[pl.ds(base, _BPW)])


@jax.jit
def kernel(users, tours, user_table, tour_table, W, b):
    wu = W[0, :EMB].reshape(EMB, 1)
    zero = jnp.zeros((1,), jnp.float32)
    wrep = jnp.broadcast_to(W[0, EMB:].reshape(EMB, 1), (EMB, _L))
    brep = jnp.broadcast_to(b, (_L,))
    tscore = _tscores_sc(tour_table.T, wrep, brep)
    uscore = _scores(user_table.T, wu, zero, N_USERS, _UCHUNK)

    run = pl.kernel(
        _gather_body,
        out_type=jax.ShapeDtypeStruct((BATCH,), jnp.float32),
        mesh=plsc.VectorSubcoreMesh(core_axis_name="c", subcore_axis_name="s"),
        compiler_params=pltpu.CompilerParams(
            needs_layout_passes=False, use_tc_tiling_on_sc=False),
        scratch_types=[
            pltpu.VMEM((_BPW,), jnp.int32),
            pltpu.VMEM((_BPW,), jnp.int32),
            pltpu.VMEM((_BPW,), jnp.float32),
            pltpu.VMEM((_BPW,), jnp.float32),
            pltpu.VMEM((_BPW,), jnp.float32),
            pltpu.SemaphoreType.DMA,
            pltpu.SemaphoreType.DMA,
        ],
    )
    out = run(users.astype(jnp.int32), tours.astype(jnp.int32), uscore, tscore)
    return out.reshape(BATCH, 1)


# speedup vs baseline: 37.3914x; 1.2771x over previous
import functools

import jax
import jax.numpy as jnp
from jax import lax
from jax.experimental import pallas as pl
from jax.experimental.pallas import tpu as pltpu
from jax.experimental.pallas import tpu_sc as plsc

BATCH = 16384
EMB = 32
N_USERS = 1000000
N_TOURS = 100000

_info = plsc.get_sparse_core_info()
_NC = _info.num_cores
_NS = _info.num_subcores
_L = _info.num_lanes
_NW = _NC * _NS
_BPW = BATCH // _NW

_UCHUNK = 131072
_TCHUNK = 51200


def _score_body(tT_ref, w_ref, b_ref, out_ref):
    out_ref[...] = jnp.sum(tT_ref[...] * w_ref[...], axis=0) + b_ref[0]


def _scores(tT, wcol, bias, n, chunk):
    grid = (n + chunk - 1) // chunk
    return pl.pallas_call(
        _score_body,
        grid=(grid,),
        in_specs=[
            pl.BlockSpec((EMB, chunk), lambda i: (0, i)),
            pl.BlockSpec((EMB, 1), lambda i: (0, 0)),
            pl.BlockSpec(memory_space=pltpu.SMEM),
        ],
        out_specs=pl.BlockSpec((chunk,), lambda i: (i,)),
        out_shape=jax.ShapeDtypeStruct((n,), jnp.float32),
    )(tT, wcol, bias)


_TGRID_CH = 16384


def _score_fused_body(uT_ref, tT_ref, wu_ref, wt_ref, b_ref,
                      uout_ref, tout_ref):
    uout_ref[...] = jnp.sum(uT_ref[...] * wu_ref[...], axis=0)
    tout_ref[...] = jnp.sum(tT_ref[...] * wt_ref[...], axis=0) + b_ref[0]


def _scores_fused(uT, tT, wu, wt, bias):
    grid = (N_USERS + _UCHUNK - 1) // _UCHUNK
    tlast = (N_TOURS + _TGRID_CH - 1) // _TGRID_CH - 1
    return pl.pallas_call(
        _score_fused_body,
        grid=(grid,),
        in_specs=[
            pl.BlockSpec((EMB, _UCHUNK), lambda i: (0, i)),
            pl.BlockSpec((EMB, _TGRID_CH), lambda i: (0, jnp.minimum(i, tlast))),
            pl.BlockSpec((EMB, 1), lambda i: (0, 0)),
            pl.BlockSpec((EMB, 1), lambda i: (0, 0)),
            pl.BlockSpec(memory_space=pltpu.SMEM),
        ],
        out_specs=[
            pl.BlockSpec((_UCHUNK,), lambda i: (i,)),
            pl.BlockSpec((_TGRID_CH,), lambda i: (jnp.minimum(i, tlast),)),
        ],
        out_shape=[jax.ShapeDtypeStruct((N_USERS,), jnp.float32),
                   jax.ShapeDtypeStruct((N_TOURS,), jnp.float32)],
    )(uT, tT, wu, wt, bias)


def _gather_body(users_hbm, tours_hbm, us_hbm, ts_hbm, out_hbm,
                 uidx, tidx, uval, tval, outv, sem_u, sem_t):
    wid = lax.axis_index("s") * _NC + lax.axis_index("c")
    base = wid * _BPW
    pltpu.sync_copy(users_hbm.at[pl.ds(base, _BPW)], uidx)
    pltpu.sync_copy(tours_hbm.at[pl.ds(base, _BPW)], tidx)
    cu = pltpu.async_copy(us_hbm.at[uidx], uval, sem_u)
    ct = pltpu.async_copy(ts_hbm.at[tidx], tval, sem_t)
    cu.wait()
    ct.wait()

    def group(g, carry):
        sl = pl.ds(g * _L, _L)
        outv[sl] = uval[sl] + tval[sl]
        return carry

    lax.fori_loop(0, _BPW // _L, group, 0)
    pltpu.sync_copy(outv, out_hbm.at[pl.ds(base, _BPW)])


@jax.jit
def kernel(users, tours, user_table, tour_table, W, b):
    wu = W[0, :EMB].reshape(EMB, 1)
    wt = W[0, EMB:].reshape(EMB, 1)
    uscore, tscore = _scores_fused(user_table.T, tour_table.T, wu, wt, b)

    run = pl.kernel(
        _gather_body,
        out_type=jax.ShapeDtypeStruct((BATCH,), jnp.float32),
        mesh=plsc.VectorSubcoreMesh(core_axis_name="c", subcore_axis_name="s"),
        compiler_params=pltpu.CompilerParams(
            needs_layout_passes=False, use_tc_tiling_on_sc=False),
        scratch_types=[
            pltpu.VMEM((_BPW,), jnp.int32),
            pltpu.VMEM((_BPW,), jnp.int32),
            pltpu.VMEM((_BPW,), jnp.float32),
            pltpu.VMEM((_BPW,), jnp.float32),
            pltpu.VMEM((_BPW,), jnp.float32),
            pltpu.SemaphoreType.DMA,
            pltpu.SemaphoreType.DMA,
        ],
    )
    out = run(users.astype(jnp.int32), tours.astype(jnp.int32), uscore, tscore)
    return out.reshape(BATCH, 1)
